# baseline (speedup 1.0000x reference)
import functools

import jax
import jax.numpy as jnp
from jax import lax
from jax.experimental import pallas as pl
from jax.experimental.pallas import tpu as pltpu
from jax.experimental.pallas import tpu_sc as plsc

NCORE = 2
NSUB = 16
NW = NCORE * NSUB
KCMAX = 512


def _cdiv(a, b):
    return (a + b - 1) // b


def _make_agg(N, NA, F, GPW, KC):
    assert GPW % 2 == 0 and GPW >= 2
    RPS = NA // NSUB
    trash = NA - N
    mesh = plsc.VectorSubcoreMesh(core_axis_name="c", subcore_axis_name="s")

    @functools.partial(
        pl.kernel,
        out_type=jax.ShapeDtypeStruct((NCORE * NA, F), jnp.float32),
        mesh=mesh,
        compiler_params=pltpu.CompilerParams(use_tc_tiling_on_sc=False),
        scratch_types=(
            [pltpu.VMEM_SHARED((NA, F), jnp.float32)]
            + [pltpu.VMEM((GPW, KC), jnp.int32)] * 2
            + [pltpu.VMEM((KC, F), jnp.float32)] * 2
            + [pltpu.SemaphoreType.DMA] * 4
        ),
    )
    def agg(ei_hbm, y_hbm, zz_hbm, out_hbm, acc, *scr):
        RG = ei_hbm.shape[1]
        LAST = RG - (NW - 1) * GPW
        sidx, didx = scr[0], scr[1]
        rows = scr[2:4]
        semG = scr[4:6]
        semS = scr[6:8]
        cid = lax.axis_index("c")
        sid = lax.axis_index("s")
        wid = cid * NSUB + sid

        @pl.when(wid < NW - 1)
        def _():
            pltpu.sync_copy(ei_hbm.at[0, pl.ds(wid * GPW, GPW)], sidx)
            pltpu.sync_copy(ei_hbm.at[1, pl.ds(wid * GPW, GPW)], didx)

        @pl.when(wid == NW - 1)
        def _():
            pltpu.sync_copy(ei_hbm.at[0, pl.ds(wid * GPW, LAST)],
                            sidx.at[pl.ds(0, LAST)])
            pltpu.sync_copy(ei_hbm.at[1, pl.ds(wid * GPW, LAST)],
                            didx.at[pl.ds(0, LAST)])
            lanes = lax.iota(jnp.int32, 16)
            PC = KC // 16

            @pl.loop(0, (GPW - LAST) * PC)
            def _(t):
                r = LAST + t // PC
                c = (t % PC) * 16
                g = t * 16 + lanes
                sidx[r, pl.ds(c, 16)] = lax.rem(g, N)
                didx[r, pl.ds(c, 16)] = N + lax.rem(g, trash)

        pltpu.sync_copy(zz_hbm.at[pl.ds(sid * RPS, RPS)],
                        acc.at[pl.ds(sid * RPS, RPS)])
        pltpu.async_copy(y_hbm.at[sidx.at[0]], rows[0], semG[0])
        plsc.subcore_barrier()

        def body(g, a):
            b = 1 - a
            pltpu.make_async_copy(
                y_hbm.at[sidx.at[0]], rows[a], semG[a]).wait()

            @pl.when(g >= 1)
            def _():
                pltpu.make_async_copy(
                    rows[b], acc.at[didx.at[0]], semS[b]).wait()

            @pl.when(g + 1 < GPW)
            def _():
                pltpu.async_copy(y_hbm.at[sidx.at[g + 1]], rows[b], semG[b])

            pltpu.async_copy(
                rows[a], acc.at[didx.at[g]], semS[a], add=True)

        @pl.loop(0, GPW // 2)
        def _(t):
            body(2 * t, 0)
            body(2 * t + 1, 1)

        pltpu.make_async_copy(
            rows[1], acc.at[didx.at[0]], semS[1]).wait()
        plsc.subcore_barrier()
        pltpu.sync_copy(acc.at[pl.ds(sid * RPS, RPS)],
                        out_hbm.at[pl.ds(cid * NA + sid * RPS, RPS)])

    return agg


def _pre_body(d, pad, x_ref, w1_ref, yaug_ref):
    x = x_ref[...]
    y = jnp.dot(x, w1_ref[d:2 * d, :], preferred_element_type=jnp.float32)
    cols = lax.broadcasted_iota(jnp.int32, (x.shape[0], pad), 1)
    extra = jnp.where(cols == 0, 1.0, 0.0).astype(jnp.float32)
    yaug_ref[...] = jnp.concatenate([y, extra], axis=1)


def _mid_body(n, na, d, f1, a_ref, x_ref, w1_ref, b1_ref, w2_ref,
              z_ref, h1a_ref):
    s = a_ref[0:n, 0:f1] + a_ref[na:na + n, 0:f1]
    deg = a_ref[0:n, f1:f1 + 1] + a_ref[na:na + n, f1:f1 + 1]
    rdeg = 1.0 / jnp.maximum(deg, 1.0)
    xa = jnp.dot(x_ref[...], w1_ref[0:d, :], preferred_element_type=jnp.float32)
    h1 = jnp.maximum(xa + s * rdeg + b1_ref[...].reshape(1, -1), 0.0)
    z_ref[...] = jnp.dot(h1, w2_ref[f1:2 * f1, :],
                         preferred_element_type=jnp.float32)
    h1a_ref[...] = jnp.dot(h1, w2_ref[0:f1, :],
                           preferred_element_type=jnp.float32)


def _post_body(n, na, f1, a1_ref, a_ref, h1a_ref, b2_ref, w3_ref, b3_ref,
               out_ref):
    deg = a1_ref[0:n, f1:f1 + 1] + a1_ref[na:na + n, f1:f1 + 1]
    rdeg = 1.0 / jnp.maximum(deg, 1.0)
    s2 = a_ref[0:n, :] + a_ref[na:na + n, :]
    mean2 = s2 * rdeg
    h2 = jnp.maximum(h1a_ref[...] + mean2 + b2_ref[...].reshape(1, -1), 0.0)
    out_ref[...] = (jnp.dot(h2, w3_ref[...], preferred_element_type=jnp.float32)
                    + b3_ref[...].reshape(1, 1))


def kernel(x, edge_index, W1, b1, W2, b2, W3, b3):
    N, D = x.shape
    E = edge_index.shape[1]
    F1 = W1.shape[1]
    F2 = W2.shape[1]
    KC1, KC2 = 256, 512
    assert E % KC1 == 0 and E % KC2 == 0
    GPW1 = _cdiv(E // KC1, NW * 2) * 2
    GPW2 = _cdiv(E // KC2, NW * 2) * 2
    NA = _cdiv(N + 1, NSUB * 8) * NSUB * 8

    ei1 = edge_index.reshape(2, E // KC1, KC1)
    ei2 = edge_index.reshape(2, E // KC2, KC2)

    FA = F1 + 8

    zz1 = jnp.zeros((NA, FA), jnp.float32)
    zz2 = jnp.zeros((NA, F2), jnp.float32)

    yaug = pl.pallas_call(
        functools.partial(_pre_body, D, FA - F1),
        out_shape=jax.ShapeDtypeStruct((N, FA), jnp.float32),
    )(x, W1)

    r1 = _make_agg(N, NA, FA, GPW1, KC1)(ei1, yaug, zz1)

    z, h1a = pl.pallas_call(
        functools.partial(_mid_body, N, NA, D, F1),
        out_shape=(
            jax.ShapeDtypeStruct((N, F2), jnp.float32),
            jax.ShapeDtypeStruct((N, F2), jnp.float32),
        ),
    )(r1, x, W1, b1, W2)

    r2 = _make_agg(N, NA, F2, GPW2, KC2)(ei2, z, zz2)

    out = pl.pallas_call(
        functools.partial(_post_body, N, NA, F1),
        out_shape=jax.ShapeDtypeStruct((N, 1), jnp.float32),
    )(r1, r2, h1a, b2, W3, b3)

    return out


# speedup vs baseline: 1.0470x; 1.0470x over previous
import functools

import jax
import jax.numpy as jnp
from jax import lax
from jax.experimental import pallas as pl
from jax.experimental.pallas import tpu as pltpu
from jax.experimental.pallas import tpu_sc as plsc

NCORE = 2
NSUB = 16
NW = NCORE * NSUB
KCMAX = 512


def _cdiv(a, b):
    return (a + b - 1) // b


def _make_agg(N, NA, F, GPW, KC, with_deg=False):
    assert GPW % 2 == 0 and GPW >= 2
    RPS = NA // NSUB
    trash = NA - N
    mesh = plsc.VectorSubcoreMesh(core_axis_name="c", subcore_axis_name="s")

    out_t = jax.ShapeDtypeStruct((NCORE * NA, F), jnp.float32)
    if with_deg:
        out_t = (out_t, jax.ShapeDtypeStruct((NW, NA), jnp.float32))

    @functools.partial(
        pl.kernel,
        out_type=out_t,
        mesh=mesh,
        compiler_params=pltpu.CompilerParams(
            use_tc_tiling_on_sc=False, needs_layout_passes=not with_deg),
        scratch_types=(
            [pltpu.VMEM_SHARED((NA, F), jnp.float32)]
            + [pltpu.VMEM((GPW, KC), jnp.int32)] * 2
            + [pltpu.VMEM((KC, F), jnp.float32)] * 2
            + ([pltpu.VMEM((NA,), jnp.float32)] if with_deg else [])
            + [pltpu.SemaphoreType.DMA] * 4
        ),
    )
    def agg(ei_hbm, y_hbm, zz_hbm, *rest):
        if with_deg:
            out_hbm, deg_hbm, acc = rest[0], rest[1], rest[2]
            sidx, didx, r0, r1, degb = rest[3:8]
            sems = rest[8:12]
        else:
            out_hbm, acc = rest[0], rest[1]
            sidx, didx, r0, r1 = rest[2:6]
            degb = None
            sems = rest[6:10]
        rows = (r0, r1)
        semG = sems[0:2]
        semS = sems[2:4]
        RG = ei_hbm.shape[1]
        LAST = RG - (NW - 1) * GPW
        cid = lax.axis_index("c")
        sid = lax.axis_index("s")
        wid = cid * NSUB + sid

        @pl.when(wid < NW - 1)
        def _():
            pltpu.sync_copy(ei_hbm.at[0, pl.ds(wid * GPW, GPW)], sidx)
            pltpu.sync_copy(ei_hbm.at[1, pl.ds(wid * GPW, GPW)], didx)

        @pl.when(wid == NW - 1)
        def _():
            pltpu.sync_copy(ei_hbm.at[0, pl.ds(wid * GPW, LAST)],
                            sidx.at[pl.ds(0, LAST)])
            pltpu.sync_copy(ei_hbm.at[1, pl.ds(wid * GPW, LAST)],
                            didx.at[pl.ds(0, LAST)])
            lanes = lax.iota(jnp.int32, 16)
            PC = KC // 16

            @pl.loop(0, (GPW - LAST) * PC)
            def _(t):
                r = LAST + t // PC
                c = (t % PC) * 16
                g = t * 16 + lanes
                sidx[r, pl.ds(c, 16)] = lax.rem(g, N)
                didx[r, pl.ds(c, 16)] = N + lax.rem(g, trash)

        pltpu.sync_copy(zz_hbm.at[pl.ds(sid * RPS, RPS)],
                        acc.at[pl.ds(sid * RPS, RPS)])
        pltpu.async_copy(y_hbm.at[sidx.at[0]], rows[0], semG[0])

        if with_deg:
            zero16 = jnp.zeros((16,), jnp.float32)
            one16 = jnp.ones((16,), jnp.float32)
            PC = KC // 16

            @pl.loop(0, NA // 16)
            def _(t):
                degb[pl.ds(t * 16, 16)] = zero16

            @pl.loop(0, GPW * PC)
            def _(t):
                idx = didx[t // PC, pl.ds((t % PC) * 16, 16)]
                plsc.addupdate_scatter(degb, [idx], one16)

        plsc.subcore_barrier()

        def body(g, a):
            b = 1 - a
            pltpu.make_async_copy(
                y_hbm.at[sidx.at[0]], rows[a], semG[a]).wait()

            @pl.when(g >= 1)
            def _():
                pltpu.make_async_copy(
                    rows[b], acc.at[didx.at[0]], semS[b]).wait()

            @pl.when(g + 1 < GPW)
            def _():
                pltpu.async_copy(y_hbm.at[sidx.at[g + 1]], rows[b], semG[b])

            pltpu.async_copy(
                rows[a], acc.at[didx.at[g]], semS[a], add=True)

        @pl.loop(0, GPW // 2)
        def _(t):
            body(2 * t, 0)
            body(2 * t + 1, 1)

        pltpu.make_async_copy(
            rows[1], acc.at[didx.at[0]], semS[1]).wait()
        if with_deg:
            pltpu.sync_copy(degb, deg_hbm.at[wid])
        plsc.subcore_barrier()
        pltpu.sync_copy(acc.at[pl.ds(sid * RPS, RPS)],
                        out_hbm.at[pl.ds(cid * NA + sid * RPS, RPS)])

    return agg


def _pre_body(d, x_ref, w1_ref, y_ref):
    y_ref[...] = jnp.dot(x_ref[...], w1_ref[d:2 * d, :],
                         preferred_element_type=jnp.float32)


def _rdeg(d_ref, n):
    ones_col = jnp.ones((NW, 1), jnp.float32)
    deg = lax.dot_general(d_ref[...], ones_col, (((0,), (0,)), ((), ())),
                          preferred_element_type=jnp.float32)
    return 1.0 / jnp.maximum(deg[0:n], 1.0)


def _mid_body(n, na, d, f1, a_ref, deg_ref, x_ref, w1_ref, b1_ref, w2_ref,
              z_ref, h1a_ref):
    s = a_ref[0:n, :] + a_ref[na:na + n, :]
    rdeg = _rdeg(deg_ref, n)
    xa = jnp.dot(x_ref[...], w1_ref[0:d, :], preferred_element_type=jnp.float32)
    h1 = jnp.maximum(xa + s * rdeg + b1_ref[...].reshape(1, -1), 0.0)
    z_ref[...] = jnp.dot(h1, w2_ref[f1:2 * f1, :],
                         preferred_element_type=jnp.float32)
    h1a_ref[...] = jnp.dot(h1, w2_ref[0:f1, :],
                           preferred_element_type=jnp.float32)


def _post_body(n, na, a_ref, deg_ref, h1a_ref, b2_ref, w3_ref, b3_ref,
               out_ref):
    s2 = a_ref[0:n, :] + a_ref[na:na + n, :]
    mean2 = s2 * _rdeg(deg_ref, n)
    h2 = jnp.maximum(h1a_ref[...] + mean2 + b2_ref[...].reshape(1, -1), 0.0)
    out_ref[...] = (jnp.dot(h2, w3_ref[...], preferred_element_type=jnp.float32)
                    + b3_ref[...].reshape(1, 1))


def kernel(x, edge_index, W1, b1, W2, b2, W3, b3):
    N, D = x.shape
    E = edge_index.shape[1]
    F1 = W1.shape[1]
    F2 = W2.shape[1]
    KC1, KC2 = 256, 512
    assert E % KC1 == 0 and E % KC2 == 0
    GPW1 = _cdiv(E // KC1, NW * 2) * 2
    GPW2 = _cdiv(E // KC2, NW * 2) * 2
    NA = _cdiv(N + 1, NSUB * 8) * NSUB * 8

    ei1 = edge_index.reshape(2, E // KC1, KC1)
    ei2 = edge_index.reshape(2, E // KC2, KC2)

    FA = F1

    zz1 = jnp.zeros((NA, FA), jnp.float32)
    zz2 = jnp.zeros((NA, F2), jnp.float32)

    yaug = pl.pallas_call(
        functools.partial(_pre_body, D),
        out_shape=jax.ShapeDtypeStruct((N, FA), jnp.float32),
    )(x, W1)

    r1, deg = _make_agg(N, NA, FA, GPW1, KC1, with_deg=True)(ei1, yaug, zz1)

    z, h1a = pl.pallas_call(
        functools.partial(_mid_body, N, NA, D, F1),
        out_shape=(
            jax.ShapeDtypeStruct((N, F2), jnp.float32),
            jax.ShapeDtypeStruct((N, F2), jnp.float32),
        ),
    )(r1, deg, x, W1, b1, W2)

    r2 = _make_agg(N, NA, F2, GPW2, KC2)(ei2, z, zz2)

    out = pl.pallas_call(
        functools.partial(_post_body, N, NA),
        out_shape=jax.ShapeDtypeStruct((N, 1), jnp.float32),
    )(r2, deg, h1a, b2, W3, b3)

    return out
